# baseline (speedup 1.0000x reference)
import functools

import jax
import jax.numpy as jnp
from jax import lax
from jax.experimental import pallas as pl
from jax.experimental.pallas import tpu as pltpu
from jax.experimental.pallas import tpu_sc as plsc

NUM_SEG = 512
D_FEAT = 128
N_ROWS = 100000
NUM_CORES = 2
NUM_TILES = 16
NUM_WORKERS = NUM_CORES * NUM_TILES
CHUNK = 125
CH_PAD = 128
NUM_CHUNKS = N_ROWS // CHUNK
CHUNKS_PER_W = NUM_CHUNKS // NUM_WORKERS
NUM_PAIRS = CHUNKS_PER_W // 2
LANES = 16

_mesh = plsc.VectorSubcoreMesh(core_axis_name="c", subcore_axis_name="s")


@functools.partial(
    pl.kernel,
    out_type=(
        jax.ShapeDtypeStruct((NUM_CORES * NUM_SEG, D_FEAT), jnp.float32),
        jax.ShapeDtypeStruct((NUM_WORKERS, NUM_SEG), jnp.float32),
    ),
    mesh=_mesh,
    compiler_params=pltpu.CompilerParams(use_tc_tiling_on_sc=False,
                                         needs_layout_passes=False),
    scratch_types=[
        pltpu.VMEM((CH_PAD, D_FEAT), jnp.float32),
        pltpu.VMEM((CH_PAD, D_FEAT), jnp.float32),
        pltpu.VMEM((CH_PAD,), jnp.int32),
        pltpu.VMEM((CH_PAD,), jnp.int32),
        pltpu.VMEM((NUM_SEG,), jnp.float32),
        pltpu.VMEM_SHARED((NUM_SEG, D_FEAT), jnp.float32),
        pltpu.SemaphoreType.DMA,
        pltpu.SemaphoreType.DMA,
        pltpu.SemaphoreType.DMA,
        pltpu.SemaphoreType.DMA,
    ],
)
def _sc_segment_sum(x_hbm, b2_hbm, zf_hbm, psums_hbm, pcnts_hbm,
                    rows_a, rows_b, idx_a, idx_b, cnts, acc,
                    lsem_a, lsem_b, ssem_a, ssem_b):
    cid = lax.axis_index("c")
    sid = lax.axis_index("s")
    w = cid * NUM_TILES + sid
    rows_per_tile = NUM_SEG // NUM_TILES
    seg_base = sid * rows_per_tile

    pltpu.sync_copy(zf_hbm.at[pl.ds(0, CH_PAD - CHUNK)],
                    rows_a.at[pl.ds(CHUNK, CH_PAD - CHUNK)])
    pltpu.sync_copy(zf_hbm.at[pl.ds(0, CH_PAD - CHUNK)],
                    rows_b.at[pl.ds(CHUNK, CH_PAD - CHUNK)])
    pltpu.sync_copy(zf_hbm.at[pl.ds(seg_base, rows_per_tile)],
                    acc.at[pl.ds(seg_base, rows_per_tile)])
    zero16 = jnp.zeros((LANES,), jnp.float32)

    def zbody(r, carry):
        cnts[pl.ds(r * LANES, LANES)] = zero16
        return carry

    lax.fori_loop(0, NUM_SEG // LANES, zbody, 0)
    plsc.subcore_barrier()

    one16 = jnp.ones((LANES,), jnp.float32)
    tail_mask = lax.iota(jnp.int32, LANES) < (CHUNK % LANES)

    def load(c, buf, idx, sem):
        d0 = pltpu.async_copy(
            x_hbm.at[pl.ds((w * CHUNKS_PER_W + c) * CHUNK, CHUNK)],
            buf.at[pl.ds(0, CHUNK)], sem)
        d1 = pltpu.async_copy(b2_hbm.at[w * CHUNKS_PER_W + c], idx, sem)
        return d0, d1

    def wait(d):
        d[0].wait()
        d[1].wait()

    def scatter(buf, idx, sem):
        return pltpu.async_copy(buf, acc.at[idx], sem, add=True)

    def count(idx):
        for k in range(CHUNK // LANES):
            seg = idx[pl.ds(k * LANES, LANES)]
            plsc.addupdate_scatter(cnts, [seg], one16)
        seg = idx[pl.ds((CHUNK // LANES) * LANES, LANES)]
        plsc.addupdate_scatter(cnts, [seg], one16, mask=tail_mask)

    wait(load(0, rows_a, idx_a, lsem_a))

    def pair(t, carry):
        c0 = 2 * t
        lb = load(c0 + 1, rows_b, idx_b, lsem_b)
        sa = scatter(rows_a, idx_a, ssem_a)
        count(idx_a)
        wait(lb)
        sb = scatter(rows_b, idx_b, ssem_b)
        count(idx_b)
        sa.wait()
        la = load(c0 + 2, rows_a, idx_a, lsem_a)
        sb.wait()
        wait(la)
        return carry

    lax.fori_loop(0, NUM_PAIRS, pair, 0)

    st = scatter(rows_a, idx_a, ssem_a)
    count(idx_a)
    st.wait()
    plsc.subcore_barrier()

    out_base = cid * NUM_SEG + seg_base
    pltpu.sync_copy(acc.at[pl.ds(seg_base, rows_per_tile)],
                    psums_hbm.at[pl.ds(out_base, rows_per_tile)])
    pltpu.sync_copy(cnts, pcnts_hbm.at[w])


XCOPY_BLK = 4000


def _combine_body(ps_ref, pc_ref, xb_ref, xo_ref, pr_ref):
    xo_ref[...] = xb_ref[...]

    @pl.when(pl.program_id(0) == 0)
    def _():
        s = ps_ref[0] + ps_ref[1]
        c = jnp.sum(pc_ref[...], axis=0)
        scale = 1.0 / (jnp.maximum(c, 1.0) * jnp.sqrt(c + 1e-6))
        pr_ref[...] = s * scale[:, None]


def kernel(x, batch):
    b2 = jnp.pad(batch.reshape(NUM_CHUNKS, CHUNK),
                 ((0, 0), (0, CH_PAD - CHUNK)))
    zf = jnp.zeros((NUM_SEG, D_FEAT), jnp.float32)
    psums, pcnts = _sc_segment_sum(x, b2, zf)
    x_out, protein_repr = pl.pallas_call(
        _combine_body,
        grid=(N_ROWS // XCOPY_BLK,),
        in_specs=[
            pl.BlockSpec((NUM_CORES, NUM_SEG, D_FEAT), lambda i: (0, 0, 0)),
            pl.BlockSpec((NUM_WORKERS, NUM_SEG), lambda i: (0, 0)),
            pl.BlockSpec((XCOPY_BLK, D_FEAT), lambda i: (i, 0)),
        ],
        out_specs=[
            pl.BlockSpec((XCOPY_BLK, D_FEAT), lambda i: (i, 0)),
            pl.BlockSpec((NUM_SEG, D_FEAT), lambda i: (0, 0)),
        ],
        out_shape=[
            jax.ShapeDtypeStruct((N_ROWS, D_FEAT), jnp.float32),
            jax.ShapeDtypeStruct((NUM_SEG, D_FEAT), jnp.float32),
        ],
    )(psums.reshape(NUM_CORES, NUM_SEG, D_FEAT), pcnts, x)
    return x_out, protein_repr


# speedup vs baseline: 1.4126x; 1.4126x over previous
import functools

import jax
import jax.numpy as jnp
from jax import lax
from jax.experimental import pallas as pl
from jax.experimental.pallas import tpu as pltpu
from jax.experimental.pallas import tpu_sc as plsc

NUM_SEG = 512
D_FEAT = 128
N_ROWS = 100000
NUM_CORES = 2
NUM_TILES = 16
NUM_WORKERS = NUM_CORES * NUM_TILES
CHUNK = 125
CH_PAD = 128
NUM_CHUNKS = N_ROWS // CHUNK
CHUNKS_PER_W = NUM_CHUNKS // NUM_WORKERS
NUM_PAIRS = CHUNKS_PER_W // 2
LANES = 16

_mesh = plsc.VectorSubcoreMesh(core_axis_name="c", subcore_axis_name="s")


@functools.partial(
    pl.kernel,
    out_type=(
        jax.ShapeDtypeStruct((N_ROWS, D_FEAT), jnp.float32),
        jax.ShapeDtypeStruct((NUM_CORES * NUM_SEG, D_FEAT), jnp.float32),
        jax.ShapeDtypeStruct((NUM_WORKERS, NUM_SEG), jnp.float32),
    ),
    mesh=_mesh,
    compiler_params=pltpu.CompilerParams(use_tc_tiling_on_sc=False,
                                         needs_layout_passes=False),
    scratch_types=[
        pltpu.VMEM((CH_PAD, D_FEAT), jnp.float32),
        pltpu.VMEM((CH_PAD, D_FEAT), jnp.float32),
        pltpu.VMEM((CH_PAD,), jnp.int32),
        pltpu.VMEM((CH_PAD,), jnp.int32),
        pltpu.VMEM((NUM_SEG,), jnp.float32),
        pltpu.VMEM_SHARED((NUM_SEG, D_FEAT), jnp.float32),
        pltpu.SemaphoreType.DMA,
        pltpu.SemaphoreType.DMA,
        pltpu.SemaphoreType.DMA,
        pltpu.SemaphoreType.DMA,
        pltpu.SemaphoreType.DMA,
        pltpu.SemaphoreType.DMA,
    ],
)
def _sc_segment_sum(x_hbm, b2_hbm, zf_hbm, xout_hbm, psums_hbm, pcnts_hbm,
                    rows_a, rows_b, idx_a, idx_b, cnts, acc,
                    lsem_a, lsem_b, ssem_a, ssem_b, wsem_a, wsem_b):
    cid = lax.axis_index("c")
    sid = lax.axis_index("s")
    w = cid * NUM_TILES + sid
    rows_per_tile = NUM_SEG // NUM_TILES
    seg_base = sid * rows_per_tile

    pltpu.sync_copy(zf_hbm.at[pl.ds(0, CH_PAD - CHUNK)],
                    rows_a.at[pl.ds(CHUNK, CH_PAD - CHUNK)])
    pltpu.sync_copy(zf_hbm.at[pl.ds(0, CH_PAD - CHUNK)],
                    rows_b.at[pl.ds(CHUNK, CH_PAD - CHUNK)])
    pltpu.sync_copy(zf_hbm.at[pl.ds(seg_base, rows_per_tile)],
                    acc.at[pl.ds(seg_base, rows_per_tile)])
    zero16 = jnp.zeros((LANES,), jnp.float32)

    def zbody(r, carry):
        cnts[pl.ds(r * LANES, LANES)] = zero16
        return carry

    lax.fori_loop(0, NUM_SEG // LANES, zbody, 0)
    plsc.subcore_barrier()

    one16 = jnp.ones((LANES,), jnp.float32)
    tail_mask = lax.iota(jnp.int32, LANES) < (CHUNK % LANES)

    def load(c, buf, idx, sem):
        d0 = pltpu.async_copy(
            x_hbm.at[pl.ds((w * CHUNKS_PER_W + c) * CHUNK, CHUNK)],
            buf.at[pl.ds(0, CHUNK)], sem)
        d1 = pltpu.async_copy(b2_hbm.at[w * CHUNKS_PER_W + c], idx, sem)
        return d0, d1

    def wait(d):
        d[0].wait()
        d[1].wait()

    def scatter(buf, idx, sem):
        return pltpu.async_copy(buf, acc.at[idx], sem, add=True)

    def writeback(c, buf, sem):
        return pltpu.async_copy(
            buf.at[pl.ds(0, CHUNK)],
            xout_hbm.at[pl.ds((w * CHUNKS_PER_W + c) * CHUNK, CHUNK)], sem)

    def count(idx):
        for k in range(CHUNK // LANES):
            seg = idx[pl.ds(k * LANES, LANES)]
            plsc.addupdate_scatter(cnts, [seg], one16)
        seg = idx[pl.ds((CHUNK // LANES) * LANES, LANES)]
        plsc.addupdate_scatter(cnts, [seg], one16, mask=tail_mask)

    wait(load(0, rows_a, idx_a, lsem_a))

    def pair(t, carry):
        c0 = 2 * t
        lb = load(c0 + 1, rows_b, idx_b, lsem_b)
        sa = scatter(rows_a, idx_a, ssem_a)
        wa = writeback(c0, rows_a, wsem_a)
        count(idx_a)
        wait(lb)
        sb = scatter(rows_b, idx_b, ssem_b)
        wb = writeback(c0 + 1, rows_b, wsem_b)
        count(idx_b)
        sa.wait()
        wa.wait()
        la = load(c0 + 2, rows_a, idx_a, lsem_a)
        sb.wait()
        wb.wait()
        wait(la)
        return carry

    lax.fori_loop(0, NUM_PAIRS, pair, 0)

    st = scatter(rows_a, idx_a, ssem_a)
    wt = writeback(CHUNKS_PER_W - 1, rows_a, wsem_a)
    count(idx_a)
    st.wait()
    wt.wait()
    plsc.subcore_barrier()

    out_base = cid * NUM_SEG + seg_base
    pltpu.sync_copy(acc.at[pl.ds(seg_base, rows_per_tile)],
                    psums_hbm.at[pl.ds(out_base, rows_per_tile)])
    pltpu.sync_copy(cnts, pcnts_hbm.at[w])


def _combine_body(ps_ref, pc_ref, o_ref):
    s = ps_ref[0] + ps_ref[1]
    c = jnp.sum(pc_ref[...], axis=0)
    scale = 1.0 / (jnp.maximum(c, 1.0) * jnp.sqrt(c + 1e-6))
    o_ref[...] = s * scale[:, None]


def kernel(x, batch):
    b2 = jnp.pad(batch.reshape(NUM_CHUNKS, CHUNK),
                 ((0, 0), (0, CH_PAD - CHUNK)))
    zf = jnp.zeros((NUM_SEG, D_FEAT), jnp.float32)
    x_out, psums, pcnts = _sc_segment_sum(x, b2, zf)
    protein_repr = pl.pallas_call(
        _combine_body,
        out_shape=jax.ShapeDtypeStruct((NUM_SEG, D_FEAT), jnp.float32),
    )(psums.reshape(NUM_CORES, NUM_SEG, D_FEAT), pcnts)
    return x_out, protein_repr


# speedup vs baseline: 1.4154x; 1.0019x over previous
import functools

import jax
import jax.numpy as jnp
from jax import lax
from jax.experimental import pallas as pl
from jax.experimental.pallas import tpu as pltpu
from jax.experimental.pallas import tpu_sc as plsc

NUM_SEG = 512
D_FEAT = 128
N_ROWS = 100000
NUM_CORES = 2
NUM_TILES = 16
NUM_WORKERS = NUM_CORES * NUM_TILES
CHUNK = 125
CH_PAD = 128
NUM_CHUNKS = N_ROWS // CHUNK
CHUNKS_PER_W = NUM_CHUNKS // NUM_WORKERS
NUM_PAIRS = CHUNKS_PER_W // 2
LANES = 16

_mesh = plsc.VectorSubcoreMesh(core_axis_name="c", subcore_axis_name="s")


@functools.partial(
    pl.kernel,
    out_type=(
        jax.ShapeDtypeStruct((N_ROWS, D_FEAT), jnp.float32),
        jax.ShapeDtypeStruct((NUM_CORES * NUM_SEG, D_FEAT), jnp.float32),
        jax.ShapeDtypeStruct((NUM_WORKERS, NUM_SEG), jnp.float32),
    ),
    mesh=_mesh,
    compiler_params=pltpu.CompilerParams(use_tc_tiling_on_sc=False,
                                         needs_layout_passes=False),
    scratch_types=[
        pltpu.VMEM((CH_PAD, D_FEAT), jnp.float32),
        pltpu.VMEM((CH_PAD, D_FEAT), jnp.float32),
        pltpu.VMEM((CH_PAD, D_FEAT), jnp.float32),
        pltpu.VMEM((CH_PAD,), jnp.int32),
        pltpu.VMEM((CH_PAD,), jnp.int32),
        pltpu.VMEM((CH_PAD,), jnp.int32),
        pltpu.VMEM((NUM_SEG,), jnp.float32),
        pltpu.VMEM_SHARED((NUM_SEG, D_FEAT), jnp.float32),
        pltpu.SemaphoreType.DMA,
        pltpu.SemaphoreType.DMA,
        pltpu.SemaphoreType.DMA,
        pltpu.SemaphoreType.DMA,
        pltpu.SemaphoreType.DMA,
        pltpu.SemaphoreType.DMA,
        pltpu.SemaphoreType.DMA,
        pltpu.SemaphoreType.DMA,
        pltpu.SemaphoreType.DMA,
    ],
)
def _sc_segment_sum(x_hbm, b2_hbm, zf_hbm, xout_hbm, psums_hbm, pcnts_hbm,
                    rows_a, rows_b, rows_c, idx_a, idx_b, idx_c, cnts, acc,
                    lsem_a, lsem_b, lsem_c, ssem_a, ssem_b, ssem_c,
                    wsem_a, wsem_b, wsem_c):
    cid = lax.axis_index("c")
    sid = lax.axis_index("s")
    w = cid * NUM_TILES + sid
    rows_per_tile = NUM_SEG // NUM_TILES
    seg_base = sid * rows_per_tile

    pltpu.sync_copy(zf_hbm.at[pl.ds(0, CH_PAD - CHUNK)],
                    rows_a.at[pl.ds(CHUNK, CH_PAD - CHUNK)])
    pltpu.sync_copy(zf_hbm.at[pl.ds(0, CH_PAD - CHUNK)],
                    rows_b.at[pl.ds(CHUNK, CH_PAD - CHUNK)])
    pltpu.sync_copy(zf_hbm.at[pl.ds(0, CH_PAD - CHUNK)],
                    rows_c.at[pl.ds(CHUNK, CH_PAD - CHUNK)])
    pltpu.sync_copy(zf_hbm.at[pl.ds(seg_base, rows_per_tile)],
                    acc.at[pl.ds(seg_base, rows_per_tile)])
    zero16 = jnp.zeros((LANES,), jnp.float32)

    def zbody(r, carry):
        cnts[pl.ds(r * LANES, LANES)] = zero16
        return carry

    lax.fori_loop(0, NUM_SEG // LANES, zbody, 0)
    plsc.subcore_barrier()

    one16 = jnp.ones((LANES,), jnp.float32)
    tail_mask = lax.iota(jnp.int32, LANES) < (CHUNK % LANES)

    def load(c, buf, idx, sem):
        d0 = pltpu.async_copy(
            x_hbm.at[pl.ds((w * CHUNKS_PER_W + c) * CHUNK, CHUNK)],
            buf.at[pl.ds(0, CHUNK)], sem)
        d1 = pltpu.async_copy(b2_hbm.at[w * CHUNKS_PER_W + c], idx, sem)
        return d0, d1

    def wait(d):
        d[0].wait()
        d[1].wait()

    def scatter(buf, idx, sem):
        return pltpu.async_copy(buf, acc.at[idx], sem, add=True)

    def writeback(c, buf, sem):
        return pltpu.async_copy(
            buf.at[pl.ds(0, CHUNK)],
            xout_hbm.at[pl.ds((w * CHUNKS_PER_W + c) * CHUNK, CHUNK)], sem)

    def count(idx):
        for k in range(CHUNK // LANES):
            seg = idx[pl.ds(k * LANES, LANES)]
            plsc.addupdate_scatter(cnts, [seg], one16)
        seg = idx[pl.ds((CHUNK // LANES) * LANES, LANES)]
        plsc.addupdate_scatter(cnts, [seg], one16, mask=tail_mask)

    NUM_TRIPLES = 8
    wait(load(0, rows_a, idx_a, lsem_a))
    lb0 = load(1, rows_b, idx_b, lsem_b)

    def process(c, buf, idx, ssem, wsem):
        s = scatter(buf, idx, ssem)
        wb = writeback(c, buf, wsem)
        count(idx)
        return s, wb

    def triple(t, carry):
        c0 = 3 * t
        lc = load(c0 + 2, rows_c, idx_c, lsem_c)
        sa, wa = process(c0, rows_a, idx_a, ssem_a, wsem_a)
        wait((pltpu.make_async_copy(
            x_hbm.at[pl.ds(((w * CHUNKS_PER_W) + c0 + 1) * CHUNK, CHUNK)],
            rows_b.at[pl.ds(0, CHUNK)], lsem_b),
            pltpu.make_async_copy(
                b2_hbm.at[w * CHUNKS_PER_W + c0 + 1], idx_b, lsem_b)))
        sb, wb = process(c0 + 1, rows_b, idx_b, ssem_b, wsem_b)
        sa.wait()
        wa.wait()
        la = load(c0 + 3, rows_a, idx_a, lsem_a)
        wait(lc)
        sc, wc = process(c0 + 2, rows_c, idx_c, ssem_c, wsem_c)
        sb.wait()
        wb.wait()

        @pl.when(t < NUM_TRIPLES - 1)
        def _():
            d = load(c0 + 4, rows_b, idx_b, lsem_b)
            del d
        sc.wait()
        wc.wait()
        wait(la)
        return carry

    lax.fori_loop(0, NUM_TRIPLES, triple, 0)

    st, wt = process(CHUNKS_PER_W - 1, rows_a, idx_a, ssem_a, wsem_a)
    st.wait()
    wt.wait()
    plsc.subcore_barrier()

    out_base = cid * NUM_SEG + seg_base
    pltpu.sync_copy(acc.at[pl.ds(seg_base, rows_per_tile)],
                    psums_hbm.at[pl.ds(out_base, rows_per_tile)])
    pltpu.sync_copy(cnts, pcnts_hbm.at[w])


def _combine_body(ps_ref, pc_ref, o_ref):
    s = ps_ref[0] + ps_ref[1]
    c = jnp.sum(pc_ref[...], axis=0)
    scale = 1.0 / (jnp.maximum(c, 1.0) * jnp.sqrt(c + 1e-6))
    o_ref[...] = s * scale[:, None]


def kernel(x, batch):
    b2 = jnp.pad(batch.reshape(NUM_CHUNKS, CHUNK),
                 ((0, 0), (0, CH_PAD - CHUNK)))
    zf = jnp.zeros((NUM_SEG, D_FEAT), jnp.float32)
    x_out, psums, pcnts = _sc_segment_sum(x, b2, zf)
    protein_repr = pl.pallas_call(
        _combine_body,
        out_shape=jax.ShapeDtypeStruct((NUM_SEG, D_FEAT), jnp.float32),
    )(psums.reshape(NUM_CORES, NUM_SEG, D_FEAT), pcnts)
    return x_out, protein_repr
